# baseline (device time: 19090 ns/iter reference)
import jax
import jax.numpy as jnp
from jax import lax
from jax.experimental import pallas as pl
from jax.experimental.pallas import tpu as pltpu

N_DEV = 4
B, Sq, Skv, Hq, Dh = 2, 128, 128, 16, 64
H_LOC = Hq // N_DEV
D_LOC = H_LOC * Dh
D_MODEL = 512
NEG_INF = -1e9


def _body(x_ref, wq_ref, k_hbm, v_hbm, wo_ref, out_ref,
          k_loc, v_loc, comm_ref, local_sems, send_sems, recv_sems):
    my_pos = lax.axis_index("i")
    h0 = my_pos * H_LOC
    p_a = my_pos ^ 1
    p_b = (N_DEV - 1) - my_pos

    k_copy = pltpu.make_async_copy(
        k_hbm.at[:, :, pl.ds(h0, H_LOC), :], k_loc, local_sems.at[0])
    v_copy = pltpu.make_async_copy(
        v_hbm.at[:, :, pl.ds(h0, H_LOC), :], v_loc, local_sems.at[1])
    k_copy.start()
    v_copy.start()

    barrier_sem = pltpu.get_barrier_semaphore()
    for nbr in (p_a, p_b):
        pl.semaphore_signal(
            barrier_sem, inc=1,
            device_id=(nbr,), device_id_type=pl.DeviceIdType.MESH,
        )
    pl.semaphore_wait(barrier_sem, 2)

    qb = lax.broadcasted_iota(jnp.int32, (Sq, Skv), 0) // 64
    kb = lax.broadcasted_iota(jnp.int32, (Sq, Skv), 1) // 64
    mask = (qb == kb) | (kb == 0) | (lax.rem(qb + kb, 3) == 0)

    def partial_for_batch(b):
        q_b = jnp.dot(x_ref[b], wq_ref[...],
                      preferred_element_type=jnp.float32) * 0.125
        ctx_h = []
        for h in range(H_LOC):
            q = q_b[:, h * Dh:(h + 1) * Dh]
            k = k_loc[b, :, h, :]
            s = lax.dot_general(
                q, k, (((1,), (1,)), ((), ())),
                preferred_element_type=jnp.float32)
            e = jnp.where(mask, jnp.exp(s), 0.0)
            w = e / jnp.sum(e, axis=-1, keepdims=True)
            ctx_h.append(jnp.dot(w, v_loc[b, :, h, :],
                                 preferred_element_type=jnp.float32))
        ctx_b = jnp.concatenate(ctx_h, axis=1)
        return jnp.dot(ctx_b, wo_ref[...],
                       preferred_element_type=jnp.float32)

    def exchange(half, partner, slot):
        return pltpu.make_async_remote_copy(
            src_ref=out_ref.at[half],
            dst_ref=comm_ref.at[slot],
            send_sem=send_sems.at[slot],
            recv_sem=recv_sems.at[slot],
            device_id=(partner,),
            device_id_type=pl.DeviceIdType.MESH,
        )

    k_copy.wait()
    v_copy.wait()

    out_ref[0] = partial_for_batch(0)
    r1h0 = exchange(0, p_a, 0)
    r1h0.start()

    out_ref[1] = partial_for_batch(1)
    r1h1 = exchange(1, p_b, 1)
    r1h1.start()

    r1h0.wait()
    out_ref[0] = out_ref[0] + comm_ref[0]
    r2h0 = exchange(0, p_b, 2)
    r2h0.start()

    r1h1.wait()
    out_ref[1] = out_ref[1] + comm_ref[1]
    r2h1 = exchange(1, p_a, 3)
    r2h1.start()

    r2h0.wait()
    out_ref[0] = out_ref[0] + comm_ref[2]
    r2h1.wait()
    out_ref[1] = out_ref[1] + comm_ref[3]


def kernel(x, Wq, K_ext, V_ext, Wo):
    return pl.pallas_call(
        _body,
        out_shape=jax.ShapeDtypeStruct((B, Sq, D_MODEL), jnp.float32),
        in_specs=[
            pl.BlockSpec(memory_space=pltpu.VMEM),
            pl.BlockSpec(memory_space=pltpu.VMEM),
            pl.BlockSpec(memory_space=pltpu.MemorySpace.HBM),
            pl.BlockSpec(memory_space=pltpu.MemorySpace.HBM),
            pl.BlockSpec(memory_space=pltpu.VMEM),
        ],
        out_specs=pl.BlockSpec(memory_space=pltpu.VMEM),
        scratch_shapes=[
            pltpu.VMEM((B, Skv, H_LOC, Dh), jnp.float32),
            pltpu.VMEM((B, Skv, H_LOC, Dh), jnp.float32),
            pltpu.VMEM((4, Sq, D_MODEL), jnp.float32),
            pltpu.SemaphoreType.DMA((2,)),
            pltpu.SemaphoreType.DMA((4,)),
            pltpu.SemaphoreType.DMA((4,)),
        ],
        compiler_params=pltpu.CompilerParams(collective_id=0),
    )(x, Wq, K_ext, V_ext, Wo)
